# Initial kernel scaffold; baseline (speedup 1.0000x reference)
import jax
import jax.numpy as jnp
from jax.experimental import pallas as pl


def kernel(student_ids, questions, responses, theta_table, alpha_raw, beta_base, beta_gaps):
    raise NotImplementedError("write your pallas kernel here")


# speedup vs baseline: 3.9277x; 3.9277x over previous
import functools

import jax
import jax.numpy as jnp
from jax import lax
from jax.experimental import pallas as pl
from jax.experimental.pallas import tpu as pltpu
from jax.experimental.pallas import tpu_sc as plsc

_B, _S = 4096, 200
_N = _B * _S
_D = 16
_K = 5
_NC, _NS = 2, 16
_NW = _NC * _NS
_BPW = _N // _NW
_IDXW = 128
_CH = 1024
_CHB = _CH // _IDXW
_NSTEP = _BPW // _CH


def _gather_body(sid_hbm, qid_hbm, theta_hbm, alpha_hbm, btab_hbm,
                 theta_out, araw_out, bcol_out,
                 sidx_v, qidx_v, throws_v, arows_v, brows_v, sem):
    wid = lax.axis_index("s") * _NC + lax.axis_index("c")

    def step(i, _):
        idx_row = wid * (_BPW // _IDXW) + i * _CHB
        base = wid * _BPW + i * _CH
        pltpu.sync_copy(sid_hbm.at[pl.ds(idx_row, _CHB)], sidx_v)
        pltpu.sync_copy(qid_hbm.at[pl.ds(idx_row, _CHB)], qidx_v)
        cps = []
        for j in range(_CHB):
            dst = pl.ds(j * _IDXW, _IDXW)
            cps.append(pltpu.async_copy(
                theta_hbm.at[sidx_v.at[j]], throws_v.at[dst], sem))
            cps.append(pltpu.async_copy(
                alpha_hbm.at[qidx_v.at[j]], arows_v.at[dst], sem))
            cps.append(pltpu.async_copy(
                btab_hbm.at[qidx_v.at[j]], brows_v.at[dst], sem))
        for cp in cps:
            cp.wait()
        pltpu.sync_copy(throws_v, theta_out.at[pl.ds(base, _CH)])
        pltpu.sync_copy(arows_v, araw_out.at[pl.ds(base, _CH)])
        pltpu.sync_copy(brows_v, bcol_out.at[pl.ds(base, _CH)])
        return 0

    lax.fori_loop(0, _NSTEP, step, 0)


def _sc_gather(sid2, qid2, theta_table, alpha_raw, btab):
    mesh = plsc.VectorSubcoreMesh(core_axis_name="c", subcore_axis_name="s")
    run = pl.kernel(
        _gather_body,
        out_type=(
            jax.ShapeDtypeStruct((_N, _D), jnp.float32),
            jax.ShapeDtypeStruct((_N, _D), jnp.float32),
            jax.ShapeDtypeStruct((_N, 8), jnp.float32),
        ),
        mesh=mesh,
        scratch_types=[
            pltpu.VMEM((_CHB, _IDXW), jnp.int32),
            pltpu.VMEM((_CHB, _IDXW), jnp.int32),
            pltpu.VMEM((_CH, _D), jnp.float32),
            pltpu.VMEM((_CH, _D), jnp.float32),
            pltpu.VMEM((_CH, 8), jnp.float32),
            pltpu.SemaphoreType.DMA,
        ],
        compiler_params=pltpu.CompilerParams(use_tc_tiling_on_sc=False),
    )
    return run(sid2, qid2, theta_table, alpha_raw, btab)


_RB = 2048


def _math_body(th_ref, ar_ref, bc_ref, alpha_ref, beta_ref, logit_ref, prob_ref):
    th = th_ref[...]
    alpha = jnp.exp(0.3 * ar_ref[...])
    alpha_ref[...] = alpha
    bc = bc_ref[...]
    b0 = bc[:, 0:1]
    g = jax.nn.softplus(bc[:, 1:4])
    b1 = b0 + g[:, 0:1]
    b2 = b1 + g[:, 1:2]
    b3 = b2 + g[:, 2:3]
    beta_ref[...] = jnp.concatenate([b0, b1, b2, b3], axis=-1)
    at = jnp.sum(alpha * th, axis=-1, keepdims=True)
    c1 = at - b0
    c2 = c1 + at - b1
    c3 = c2 + at - b2
    c4 = c3 + at - b3
    z = jnp.zeros_like(at)
    logits = jnp.concatenate([z, c1, c2, c3, c4], axis=-1)
    logit_ref[...] = logits
    m = jnp.maximum(jnp.maximum(jnp.maximum(jnp.maximum(z, c1), c2), c3), c4)
    e = jnp.exp(logits - m)
    prob_ref[...] = e / jnp.sum(e, axis=-1, keepdims=True)


def _tc_math(theta, araw, bcol):
    grid = _N // _RB
    return pl.pallas_call(
        _math_body,
        grid=(grid,),
        in_specs=[
            pl.BlockSpec((_RB, _D), lambda i: (i, 0)),
            pl.BlockSpec((_RB, _D), lambda i: (i, 0)),
            pl.BlockSpec((_RB, 8), lambda i: (i, 0)),
        ],
        out_specs=[
            pl.BlockSpec((_RB, _D), lambda i: (i, 0)),
            pl.BlockSpec((_RB, 4), lambda i: (i, 0)),
            pl.BlockSpec((_RB, _K), lambda i: (i, 0)),
            pl.BlockSpec((_RB, _K), lambda i: (i, 0)),
        ],
        out_shape=(
            jax.ShapeDtypeStruct((_N, _D), jnp.float32),
            jax.ShapeDtypeStruct((_N, 4), jnp.float32),
            jax.ShapeDtypeStruct((_N, _K), jnp.float32),
            jax.ShapeDtypeStruct((_N, _K), jnp.float32),
        ),
    )(theta, araw, bcol)


@jax.jit
def kernel(student_ids, questions, responses, theta_table, alpha_raw, beta_base, beta_gaps):
    del responses
    sid2 = student_ids.reshape(_N // _IDXW, _IDXW)
    qid2 = questions.reshape(_N // _IDXW, _IDXW)
    btab = jnp.concatenate(
        [beta_base, beta_gaps,
         jnp.zeros((beta_base.shape[0], 4), jnp.float32)], axis=1)
    theta, araw, bcol = _sc_gather(sid2, qid2, theta_table, alpha_raw, btab)
    alpha, beta, logits, probs = _tc_math(theta, araw, bcol)
    return (
        theta.reshape(_B, _S, _D),
        alpha.reshape(_B, _S, _D),
        beta.reshape(_B, _S, 4),
        logits.reshape(_B, _S, _K),
        probs.reshape(_B, _S, _K),
    )


# speedup vs baseline: 12.2865x; 3.1282x over previous
import jax
import jax.numpy as jnp
from jax import lax
from jax.experimental import pallas as pl
from jax.experimental.pallas import tpu as pltpu
from jax.experimental.pallas import tpu_sc as plsc

_B, _S = 4096, 200
_N = _B * _S
_D = 16
_K = 5
_NC, _NS = 2, 16
_NW = _NC * _NS
_RT = _S // 8
_Q1 = 100001

_BBLK = 2048


def _btab_body(bb_ref, bg_ref, out_ref):
    b0 = bb_ref[...]
    g = jax.nn.softplus(bg_ref[...])
    b1 = b0 + g[0:1, :]
    b2 = b1 + g[1:2, :]
    b3 = b2 + g[2:3, :]
    z = jnp.zeros_like(b0)
    out_ref[...] = jnp.concatenate([b0, b1, b2, b3, z, z, z, z], axis=0)


def _make_btab(beta_base, beta_gaps):
    bbT = beta_base.T
    bgT = beta_gaps.T
    grid = (_Q1 + _BBLK - 1) // _BBLK
    btT = pl.pallas_call(
        _btab_body,
        grid=(grid,),
        in_specs=[
            pl.BlockSpec((1, _BBLK), lambda i: (0, i)),
            pl.BlockSpec((3, _BBLK), lambda i: (0, i)),
        ],
        out_specs=pl.BlockSpec((8, _BBLK), lambda i: (0, i)),
        out_shape=jax.ShapeDtypeStruct((8, _Q1), jnp.float32),
    )(bbT, bgT)
    return btT.T


def _sc_body(sid_hbm, qid_hbm, th_hbm, al_hbm, bt_hbm,
             t5, a5, b4, l5, p5,
             sidx_v, qidx_v, th_v, al_v, bt_v,
             thT, alT, btT, lgT, prT, sem_g, sem_o):
    w = lax.axis_index("s") * _NC + lax.axis_index("c")
    lane = jnp.arange(16, dtype=jnp.int32)

    def step(t, _):
        ibase = (t * _NW + w) * 8
        pltpu.sync_copy(sid_hbm.at[pl.ds(ibase, 8)], sidx_v)
        pltpu.sync_copy(qid_hbm.at[pl.ds(ibase, 8)], qidx_v)
        gps = []
        for u in range(8):
            dst = pl.ds(u * 128, 128)
            gps.append(pltpu.async_copy(th_hbm.at[sidx_v.at[u]], th_v.at[dst], sem_g))
            gps.append(pltpu.async_copy(al_hbm.at[qidx_v.at[u]], al_v.at[dst], sem_g))
            gps.append(pltpu.async_copy(bt_hbm.at[qidx_v.at[u]], bt_v.at[dst], sem_g))
        for cp in gps:
            cp.wait()

        def body_u(u, _):
            s = t * 8 + u
            pbase = u * 128
            for j in range(8):
                rows = pbase + j * 16 + lane
                at = jnp.zeros((16,), jnp.float32)
                for d in range(16):
                    cols = jnp.full((16,), d, jnp.int32)
                    tv = plsc.load_gather(th_v, [rows, cols])
                    av = jnp.exp(0.3 * plsc.load_gather(al_v, [rows, cols]))
                    thT[u, d, pl.ds(j * 16, 16)] = tv
                    alT[u, d, pl.ds(j * 16, 16)] = av
                    at = at + av * tv
                bv = []
                for k in range(4):
                    cols = jnp.full((16,), k, jnp.int32)
                    b = plsc.load_gather(bt_v, [rows, cols])
                    btT[u, k, pl.ds(j * 16, 16)] = b
                    bv.append(b)
                c1 = at - bv[0]
                c2 = c1 + at - bv[1]
                c3 = c2 + at - bv[2]
                c4 = c3 + at - bv[3]
                z = jnp.zeros((16,), jnp.float32)
                m = jnp.maximum(jnp.maximum(jnp.maximum(jnp.maximum(z, c1), c2), c3), c4)
                e0 = jnp.exp(z - m)
                e1 = jnp.exp(c1 - m)
                e2 = jnp.exp(c2 - m)
                e3 = jnp.exp(c3 - m)
                e4 = jnp.exp(c4 - m)
                r = 1.0 / (e0 + e1 + e2 + e3 + e4)
                sl = pl.ds(j * 16, 16)
                lgT[0, u, sl] = z
                lgT[1, u, sl] = c1
                lgT[2, u, sl] = c2
                lgT[3, u, sl] = c3
                lgT[4, u, sl] = c4
                prT[0, u, sl] = e0 * r
                prT[1, u, sl] = e1 * r
                prT[2, u, sl] = e2 * r
                prT[3, u, sl] = e3 * r
                prT[4, u, sl] = e4 * r
            pltpu.async_copy(thT.at[u, pl.ds(0, 8)], t5.at[s, 0, w], sem_o)
            pltpu.async_copy(thT.at[u, pl.ds(8, 8)], t5.at[s, 1, w], sem_o)
            pltpu.async_copy(alT.at[u, pl.ds(0, 8)], a5.at[s, 0, w], sem_o)
            pltpu.async_copy(alT.at[u, pl.ds(8, 8)], a5.at[s, 1, w], sem_o)
            pltpu.async_copy(btT.at[u], b4.at[s, w], sem_o)
            return 0

        lax.fori_loop(0, 8, body_u, 0)
        for k in range(_K):
            pltpu.async_copy(lgT.at[k], l5.at[k, t, w], sem_o)
            pltpu.async_copy(prT.at[k], p5.at[k, t, w], sem_o)
        for _i in range(4 * 8):
            pltpu.make_async_copy(t5.at[0, 0, 0], thT.at[0, pl.ds(0, 8)], sem_o).wait()
        for _i in range(8):
            pltpu.make_async_copy(b4.at[0, 0], btT.at[0], sem_o).wait()
        for _i in range(2 * _K):
            pltpu.make_async_copy(l5.at[0, 0, 0], lgT.at[0], sem_o).wait()
        return 0

    lax.fori_loop(0, _RT, step, 0)


def _sc_run(sid_sc, qid_sc, theta_table, alpha_raw, btab):
    mesh = plsc.VectorSubcoreMesh(core_axis_name="c", subcore_axis_name="s")
    return pl.kernel(
        _sc_body,
        out_type=(
            jax.ShapeDtypeStruct((_S, 2, 32, 8, 128), jnp.float32),
            jax.ShapeDtypeStruct((_S, 2, 32, 8, 128), jnp.float32),
            jax.ShapeDtypeStruct((_S, 32, 4, 128), jnp.float32),
            jax.ShapeDtypeStruct((_K, _RT, 32, 8, 128), jnp.float32),
            jax.ShapeDtypeStruct((_K, _RT, 32, 8, 128), jnp.float32),
        ),
        mesh=mesh,
        scratch_types=[
            pltpu.VMEM((8, 128), jnp.int32),
            pltpu.VMEM((8, 128), jnp.int32),
            pltpu.VMEM((1024, 16), jnp.float32),
            pltpu.VMEM((1024, 16), jnp.float32),
            pltpu.VMEM((1024, 8), jnp.float32),
            pltpu.VMEM((8, 16, 128), jnp.float32),
            pltpu.VMEM((8, 16, 128), jnp.float32),
            pltpu.VMEM((8, 4, 128), jnp.float32),
            pltpu.VMEM((_K, 8, 128), jnp.float32),
            pltpu.VMEM((_K, 8, 128), jnp.float32),
            pltpu.SemaphoreType.DMA,
            pltpu.SemaphoreType.DMA,
        ],
        compiler_params=pltpu.CompilerParams(
            use_tc_tiling_on_sc=False, needs_layout_passes=False),
    )(sid_sc, qid_sc, theta_table, alpha_raw, btab)


@jax.jit
def kernel(student_ids, questions, responses, theta_table, alpha_raw, beta_base, beta_gaps):
    del responses
    def _prep(ids):
        return (ids.T.reshape(_RT, 8, 32, 128)
                .transpose(0, 2, 1, 3).reshape(_RT * 32 * 8, 128))
    sid_sc = _prep(student_ids)
    qid_sc = _prep(questions)
    btab = _make_btab(beta_base, beta_gaps)
    t5, a5, b4, l5, p5 = _sc_run(sid_sc, qid_sc, theta_table, alpha_raw, btab)
    theta = jnp.transpose(t5, (2, 4, 0, 1, 3)).reshape(_B, _S, _D)
    alpha = jnp.transpose(a5, (2, 4, 0, 1, 3)).reshape(_B, _S, _D)
    beta = jnp.transpose(b4, (1, 3, 0, 2)).reshape(_B, _S, 4)
    logits = jnp.transpose(l5, (2, 4, 1, 3, 0)).reshape(_B, _S, _K)
    probs = jnp.transpose(p5, (2, 4, 1, 3, 0)).reshape(_B, _S, _K)
    return (theta, alpha, beta, logits, probs)


# speedup vs baseline: 12.6021x; 1.0257x over previous
import jax
import jax.numpy as jnp
from jax import lax
from jax.experimental import pallas as pl
from jax.experimental.pallas import tpu as pltpu
from jax.experimental.pallas import tpu_sc as plsc

_B, _S = 4096, 200
_N = _B * _S
_D = 16
_K = 5
_NC, _NS = 2, 16
_NW = _NC * _NS
_RT = _S // 8
_Q1 = 100001
_HU = 4
_HP = _HU * 128

_BBLK = 2048


def _btab_body(bb_ref, bg_ref, out_ref):
    b0 = bb_ref[...]
    g = jax.nn.softplus(bg_ref[...])
    b1 = b0 + g[0:1, :]
    b2 = b1 + g[1:2, :]
    b3 = b2 + g[2:3, :]
    z = jnp.zeros_like(b0)
    out_ref[...] = jnp.concatenate([b0, b1, b2, b3, z, z, z, z], axis=0)


def _make_btab(beta_base, beta_gaps):
    grid = (_Q1 + _BBLK - 1) // _BBLK
    btT = pl.pallas_call(
        _btab_body,
        grid=(grid,),
        in_specs=[
            pl.BlockSpec((1, _BBLK), lambda i: (0, i)),
            pl.BlockSpec((3, _BBLK), lambda i: (0, i)),
        ],
        out_specs=pl.BlockSpec((8, _BBLK), lambda i: (0, i)),
        out_shape=jax.ShapeDtypeStruct((8, _Q1), jnp.float32),
    )(beta_base.T, beta_gaps.T)
    return btT.T


def _sc_body(sid_hbm, qid_hbm, th_hbm, al_hbm, bt_hbm,
             t5, a5, b4, l5, p5,
             sidx0, sidx1, qidx0, qidx1,
             thv0, thv1, alv0, alv1, btv0, btv1,
             thT0, thT1, alT0, alT1, btT0, btT1,
             lgT0, lgT1, prT0, prT1,
             semg0, semg1, semo0, semo1):
    w = lax.axis_index("s") * _NC + lax.axis_index("c")
    lane = jnp.arange(16, dtype=jnp.int32)
    cols = [jnp.full((16,), d, jnp.int32) for d in range(16)]
    SIDX = (sidx0, sidx1)
    QIDX = (qidx0, qidx1)
    THV = (thv0, thv1)
    ALV = (alv0, alv1)
    BTV = (btv0, btv1)
    THT = (thT0, thT1)
    ALT = (alT0, alT1)
    BTT = (btT0, btT1)
    LGT = (lgT0, lgT1)
    PRT = (prT0, prT1)
    SEMG = (semg0, semg1)
    SEMO = (semo0, semo1)

    def fire(rt, h, par):
        ibase = (rt * _NW + w) * 8 + h * _HU
        pltpu.sync_copy(sid_hbm.at[pl.ds(ibase, _HU)], SIDX[par])
        pltpu.sync_copy(qid_hbm.at[pl.ds(ibase, _HU)], QIDX[par])
        for u in range(_HU):
            dst = pl.ds(u * 128, 128)
            pltpu.async_copy(th_hbm.at[SIDX[par].at[u]], THV[par].at[dst], SEMG[par])
            pltpu.async_copy(al_hbm.at[QIDX[par].at[u]], ALV[par].at[dst], SEMG[par])
            pltpu.async_copy(bt_hbm.at[QIDX[par].at[u]], BTV[par].at[dst], SEMG[par])

    def drain_gathers(par):
        pltpu.make_async_copy(th_hbm.at[pl.ds(0, _HP)], THV[par], SEMG[par]).wait()
        pltpu.make_async_copy(al_hbm.at[pl.ds(0, _HP)], ALV[par], SEMG[par]).wait()
        pltpu.make_async_copy(bt_hbm.at[pl.ds(0, _HP)], BTV[par], SEMG[par]).wait()

    def drain_out(par):
        for _i in range(4 * _HU):
            pltpu.make_async_copy(t5.at[0, 0, 0], THT[par].at[0, pl.ds(0, 8)],
                                  SEMO[par]).wait()
        for _i in range(_HU):
            pltpu.make_async_copy(b4.at[0, 0], BTT[par].at[0], SEMO[par]).wait()
        for _i in range(2 * _K):
            pltpu.make_async_copy(b4.at[0, 0], LGT[par].at[0], SEMO[par]).wait()

    def compute_half(rt, h, par):
        thv, alv, btv = THV[par], ALV[par], BTV[par]
        thT, alT, btT = THT[par], ALT[par], BTT[par]
        lgT, prT = LGT[par], PRT[par]
        sem_o = SEMO[par]

        def body_u(u, _):
            s = rt * 8 + h * _HU + u
            pbase = u * 128
            for j in range(8):
                rows = pbase + j * 16 + lane
                sl = pl.ds(j * 16, 16)
                prods = []
                for d in range(16):
                    tv = plsc.load_gather(thv, [rows, cols[d]])
                    av = jnp.exp(0.3 * plsc.load_gather(alv, [rows, cols[d]]))
                    thT[u, d, sl] = tv
                    alT[u, d, sl] = av
                    prods.append(av * tv)
                while len(prods) > 1:
                    prods = [a + b for a, b in zip(prods[::2], prods[1::2])]
                at = prods[0]
                bv = []
                for k in range(4):
                    b = plsc.load_gather(btv, [rows, cols[k]])
                    btT[u, k, sl] = b
                    bv.append(b)
                c1 = at - bv[0]
                c2 = c1 + at - bv[1]
                c3 = c2 + at - bv[2]
                c4 = c3 + at - bv[3]
                z = jnp.zeros((16,), jnp.float32)
                m = jnp.maximum(jnp.maximum(jnp.maximum(jnp.maximum(z, c1), c2), c3), c4)
                e0 = jnp.exp(z - m)
                e1 = jnp.exp(c1 - m)
                e2 = jnp.exp(c2 - m)
                e3 = jnp.exp(c3 - m)
                e4 = jnp.exp(c4 - m)
                r = 1.0 / (e0 + e1 + e2 + e3 + e4)
                lgT[0, u, sl] = z
                lgT[1, u, sl] = c1
                lgT[2, u, sl] = c2
                lgT[3, u, sl] = c3
                lgT[4, u, sl] = c4
                prT[0, u, sl] = e0 * r
                prT[1, u, sl] = e1 * r
                prT[2, u, sl] = e2 * r
                prT[3, u, sl] = e3 * r
                prT[4, u, sl] = e4 * r
            pltpu.async_copy(thT.at[u, pl.ds(0, 8)], t5.at[s, 0, w], sem_o)
            pltpu.async_copy(thT.at[u, pl.ds(8, 8)], t5.at[s, 1, w], sem_o)
            pltpu.async_copy(alT.at[u, pl.ds(0, 8)], a5.at[s, 0, w], sem_o)
            pltpu.async_copy(alT.at[u, pl.ds(8, 8)], a5.at[s, 1, w], sem_o)
            pltpu.async_copy(btT.at[u], b4.at[s, w], sem_o)
            return 0

        lax.fori_loop(0, _HU, body_u, 0)
        osl = pl.ds(h * _HU, _HU)
        for k in range(_K):
            pltpu.async_copy(lgT.at[k], l5.at[k, rt, w, osl], sem_o)
            pltpu.async_copy(prT.at[k], p5.at[k, rt, w, osl], sem_o)

    fire(0, 0, 0)

    def step(rt, _):
        fire(rt, 1, 1)
        drain_gathers(0)

        @pl.when(rt > 0)
        def _():
            drain_out(0)

        compute_half(rt, 0, 0)
        fire(jnp.minimum(rt + 1, _RT - 1), 0, 0)
        drain_gathers(1)

        @pl.when(rt > 0)
        def _():
            drain_out(1)

        compute_half(rt, 1, 1)
        return 0

    lax.fori_loop(0, _RT, step, 0)
    drain_gathers(0)
    drain_out(0)
    drain_out(1)


def _sc_run(sid_sc, qid_sc, theta_table, alpha_raw, btab):
    mesh = plsc.VectorSubcoreMesh(core_axis_name="c", subcore_axis_name="s")
    f32 = jnp.float32
    return pl.kernel(
        _sc_body,
        out_type=(
            jax.ShapeDtypeStruct((_S, 2, 32, 8, 128), f32),
            jax.ShapeDtypeStruct((_S, 2, 32, 8, 128), f32),
            jax.ShapeDtypeStruct((_S, 32, 4, 128), f32),
            jax.ShapeDtypeStruct((_K, _RT, 32, 8, 128), f32),
            jax.ShapeDtypeStruct((_K, _RT, 32, 8, 128), f32),
        ),
        mesh=mesh,
        scratch_types=[
            pltpu.VMEM((_HU, 128), jnp.int32),
            pltpu.VMEM((_HU, 128), jnp.int32),
            pltpu.VMEM((_HU, 128), jnp.int32),
            pltpu.VMEM((_HU, 128), jnp.int32),
            pltpu.VMEM((_HP, 16), f32),
            pltpu.VMEM((_HP, 16), f32),
            pltpu.VMEM((_HP, 16), f32),
            pltpu.VMEM((_HP, 16), f32),
            pltpu.VMEM((_HP, 8), f32),
            pltpu.VMEM((_HP, 8), f32),
            pltpu.VMEM((_HU, 16, 128), f32),
            pltpu.VMEM((_HU, 16, 128), f32),
            pltpu.VMEM((_HU, 16, 128), f32),
            pltpu.VMEM((_HU, 16, 128), f32),
            pltpu.VMEM((_HU, 4, 128), f32),
            pltpu.VMEM((_HU, 4, 128), f32),
            pltpu.VMEM((_K, _HU, 128), f32),
            pltpu.VMEM((_K, _HU, 128), f32),
            pltpu.VMEM((_K, _HU, 128), f32),
            pltpu.VMEM((_K, _HU, 128), f32),
            pltpu.SemaphoreType.DMA,
            pltpu.SemaphoreType.DMA,
            pltpu.SemaphoreType.DMA,
            pltpu.SemaphoreType.DMA,
        ],
        compiler_params=pltpu.CompilerParams(
            use_tc_tiling_on_sc=False, needs_layout_passes=False),
    )(sid_sc, qid_sc, theta_table, alpha_raw, btab)


@jax.jit
def kernel(student_ids, questions, responses, theta_table, alpha_raw, beta_base, beta_gaps):
    del responses

    def _prep(ids):
        return (ids.T.reshape(_RT, 8, 32, 128)
                .transpose(0, 2, 1, 3).reshape(_RT * 32 * 8, 128))

    sid_sc = _prep(student_ids)
    qid_sc = _prep(questions)
    btab = _make_btab(beta_base, beta_gaps)
    t5, a5, b4, l5, p5 = _sc_run(sid_sc, qid_sc, theta_table, alpha_raw, btab)
    theta = jnp.transpose(t5, (2, 4, 0, 1, 3)).reshape(_B, _S, _D)
    alpha = jnp.transpose(a5, (2, 4, 0, 1, 3)).reshape(_B, _S, _D)
    beta = jnp.transpose(b4, (1, 3, 0, 2)).reshape(_B, _S, 4)
    logits = jnp.transpose(l5, (2, 4, 1, 3, 0)).reshape(_B, _S, _K)
    probs = jnp.transpose(p5, (2, 4, 1, 3, 0)).reshape(_B, _S, _K)
    return (theta, alpha, beta, logits, probs)


# speedup vs baseline: 17.2296x; 1.3672x over previous
import jax
import jax.numpy as jnp
from jax import lax
from jax.experimental import pallas as pl
from jax.experimental.pallas import tpu as pltpu
from jax.experimental.pallas import tpu_sc as plsc

_B, _S = 4096, 200
_N = _B * _S
_D = 16
_K = 5
_NC, _NS = 2, 16
_NW = _NC * _NS
_RT = _S // 8
_Q1 = 100001
_HU = 4
_HP = _HU * 128

_BBLK = 2048


def _btab_body(bb_ref, bg_ref, out_ref):
    b0 = bb_ref[...]
    g = jax.nn.softplus(bg_ref[...])
    b1 = b0 + g[0:1, :]
    b2 = b1 + g[1:2, :]
    b3 = b2 + g[2:3, :]
    z = jnp.zeros_like(b0)
    out_ref[...] = jnp.concatenate([b0, b1, b2, b3, z, z, z, z], axis=0)


def _make_btab(beta_base, beta_gaps):
    grid = (_Q1 + _BBLK - 1) // _BBLK
    btT = pl.pallas_call(
        _btab_body,
        grid=(grid,),
        in_specs=[
            pl.BlockSpec((1, _BBLK), lambda i: (0, i)),
            pl.BlockSpec((3, _BBLK), lambda i: (0, i)),
        ],
        out_specs=pl.BlockSpec((8, _BBLK), lambda i: (0, i)),
        out_shape=jax.ShapeDtypeStruct((8, _Q1), jnp.float32),
    )(beta_base.T, beta_gaps.T)
    return btT.T


def _sc_body(sid_hbm, qid_hbm, th_hbm, al_hbm, bt_hbm,
             t5, a5, b4, l5, p5,
             sidx0, sidx1, qidx0, qidx1,
             thv0, thv1, alv0, alv1, btv0, btv1,
             thT0, thT1, alT0, alT1, btT0, btT1,
             lgT0, lgT1, prT0, prT1,
             semg0, semg1, semo0, semo1):
    w = lax.axis_index("s") * _NC + lax.axis_index("c")
    lane = jnp.arange(16, dtype=jnp.int32)
    cols = [jnp.full((16,), d, jnp.int32) for d in range(16)]
    SIDX = (sidx0, sidx1)
    QIDX = (qidx0, qidx1)
    THV = (thv0, thv1)
    ALV = (alv0, alv1)
    BTV = (btv0, btv1)
    THT = (thT0, thT1)
    ALT = (alT0, alT1)
    BTT = (btT0, btT1)
    LGT = (lgT0, lgT1)
    PRT = (prT0, prT1)
    SEMG = (semg0, semg1)
    SEMO = (semo0, semo1)

    def fire(rt, h, par):
        ibase = (rt * _NW + w) * 8 + h * _HU
        pltpu.sync_copy(sid_hbm.at[pl.ds(ibase, _HU)], SIDX[par])
        pltpu.sync_copy(qid_hbm.at[pl.ds(ibase, _HU)], QIDX[par])
        for u in range(_HU):
            dst = pl.ds(u * 128, 128)
            pltpu.async_copy(th_hbm.at[SIDX[par].at[u]], THV[par].at[dst], SEMG[par])
            pltpu.async_copy(al_hbm.at[QIDX[par].at[u]], ALV[par].at[dst], SEMG[par])
            pltpu.async_copy(bt_hbm.at[QIDX[par].at[u]], BTV[par].at[dst], SEMG[par])

    def drain_gathers(par):
        pltpu.make_async_copy(th_hbm.at[pl.ds(0, _HP)], THV[par], SEMG[par]).wait()
        pltpu.make_async_copy(al_hbm.at[pl.ds(0, _HP)], ALV[par], SEMG[par]).wait()
        pltpu.make_async_copy(bt_hbm.at[pl.ds(0, _HP)], BTV[par], SEMG[par]).wait()

    def drain_out(par):
        for _i in range(4 * _HU):
            pltpu.make_async_copy(t5.at[0, 0, 0], THT[par].at[0, pl.ds(0, 8)],
                                  SEMO[par]).wait()
        for _i in range(_HU):
            pltpu.make_async_copy(b4.at[0, 0], BTT[par].at[0], SEMO[par]).wait()
        for _i in range(2 * _K):
            pltpu.make_async_copy(b4.at[0, 0], LGT[par].at[0], SEMO[par]).wait()

    def compute_half(rt, h, par):
        thv, alv, btv = THV[par], ALV[par], BTV[par]
        thT, alT, btT = THT[par], ALT[par], BTT[par]
        lgT, prT = LGT[par], PRT[par]
        sem_o = SEMO[par]

        @plsc.parallel_loop(0, _HU * 8, unroll=2)
        def _body(i):
            u = i // 8
            rows = i * 16 + lane
            sl = pl.ds((i % 8) * 16, 16)
            prods = []
            for d in range(16):
                tv = plsc.load_gather(thv, [rows, cols[d]])
                av = jnp.exp(0.3 * plsc.load_gather(alv, [rows, cols[d]]))
                thT[u, d, sl] = tv
                alT[u, d, sl] = av
                prods.append(av * tv)
            while len(prods) > 1:
                prods = [a + b for a, b in zip(prods[::2], prods[1::2])]
            at = prods[0]
            bv = []
            for k in range(4):
                b = plsc.load_gather(btv, [rows, cols[k]])
                btT[u, k, sl] = b
                bv.append(b)
            c1 = at - bv[0]
            c2 = c1 + at - bv[1]
            c3 = c2 + at - bv[2]
            c4 = c3 + at - bv[3]
            z = jnp.zeros((16,), jnp.float32)
            m = jnp.maximum(jnp.maximum(jnp.maximum(jnp.maximum(z, c1), c2), c3), c4)
            e0 = jnp.exp(z - m)
            e1 = jnp.exp(c1 - m)
            e2 = jnp.exp(c2 - m)
            e3 = jnp.exp(c3 - m)
            e4 = jnp.exp(c4 - m)
            r = 1.0 / (e0 + e1 + e2 + e3 + e4)
            lgT[0, u, sl] = z
            lgT[1, u, sl] = c1
            lgT[2, u, sl] = c2
            lgT[3, u, sl] = c3
            lgT[4, u, sl] = c4
            prT[0, u, sl] = e0 * r
            prT[1, u, sl] = e1 * r
            prT[2, u, sl] = e2 * r
            prT[3, u, sl] = e3 * r
            prT[4, u, sl] = e4 * r

        for u in range(_HU):
            s = rt * 8 + h * _HU + u
            pltpu.async_copy(thT.at[u, pl.ds(0, 8)], t5.at[s, 0, w], sem_o)
            pltpu.async_copy(thT.at[u, pl.ds(8, 8)], t5.at[s, 1, w], sem_o)
            pltpu.async_copy(alT.at[u, pl.ds(0, 8)], a5.at[s, 0, w], sem_o)
            pltpu.async_copy(alT.at[u, pl.ds(8, 8)], a5.at[s, 1, w], sem_o)
            pltpu.async_copy(btT.at[u], b4.at[s, w], sem_o)
        osl = pl.ds(h * _HU, _HU)
        for k in range(_K):
            pltpu.async_copy(lgT.at[k], l5.at[k, rt, w, osl], sem_o)
            pltpu.async_copy(prT.at[k], p5.at[k, rt, w, osl], sem_o)

    fire(0, 0, 0)

    def step(rt, _):
        fire(rt, 1, 1)
        drain_gathers(0)

        @pl.when(rt > 0)
        def _():
            drain_out(0)

        compute_half(rt, 0, 0)
        fire(jnp.minimum(rt + 1, _RT - 1), 0, 0)
        drain_gathers(1)

        @pl.when(rt > 0)
        def _():
            drain_out(1)

        compute_half(rt, 1, 1)
        return 0

    lax.fori_loop(0, _RT, step, 0)
    drain_gathers(0)
    drain_out(0)
    drain_out(1)


def _sc_run(sid_sc, qid_sc, theta_table, alpha_raw, btab):
    mesh = plsc.VectorSubcoreMesh(core_axis_name="c", subcore_axis_name="s")
    f32 = jnp.float32
    return pl.kernel(
        _sc_body,
        out_type=(
            jax.ShapeDtypeStruct((_S, 2, 32, 8, 128), f32),
            jax.ShapeDtypeStruct((_S, 2, 32, 8, 128), f32),
            jax.ShapeDtypeStruct((_S, 32, 4, 128), f32),
            jax.ShapeDtypeStruct((_K, _RT, 32, 8, 128), f32),
            jax.ShapeDtypeStruct((_K, _RT, 32, 8, 128), f32),
        ),
        mesh=mesh,
        scratch_types=[
            pltpu.VMEM((_HU, 128), jnp.int32),
            pltpu.VMEM((_HU, 128), jnp.int32),
            pltpu.VMEM((_HU, 128), jnp.int32),
            pltpu.VMEM((_HU, 128), jnp.int32),
            pltpu.VMEM((_HP, 16), f32),
            pltpu.VMEM((_HP, 16), f32),
            pltpu.VMEM((_HP, 16), f32),
            pltpu.VMEM((_HP, 16), f32),
            pltpu.VMEM((_HP, 8), f32),
            pltpu.VMEM((_HP, 8), f32),
            pltpu.VMEM((_HU, 16, 128), f32),
            pltpu.VMEM((_HU, 16, 128), f32),
            pltpu.VMEM((_HU, 16, 128), f32),
            pltpu.VMEM((_HU, 16, 128), f32),
            pltpu.VMEM((_HU, 4, 128), f32),
            pltpu.VMEM((_HU, 4, 128), f32),
            pltpu.VMEM((_K, _HU, 128), f32),
            pltpu.VMEM((_K, _HU, 128), f32),
            pltpu.VMEM((_K, _HU, 128), f32),
            pltpu.VMEM((_K, _HU, 128), f32),
            pltpu.SemaphoreType.DMA,
            pltpu.SemaphoreType.DMA,
            pltpu.SemaphoreType.DMA,
            pltpu.SemaphoreType.DMA,
        ],
        compiler_params=pltpu.CompilerParams(
            use_tc_tiling_on_sc=False, needs_layout_passes=False),
    )(sid_sc, qid_sc, theta_table, alpha_raw, btab)


@jax.jit
def kernel(student_ids, questions, responses, theta_table, alpha_raw, beta_base, beta_gaps):
    del responses

    def _prep(ids):
        return (ids.T.reshape(_RT, 8, 32, 128)
                .transpose(0, 2, 1, 3).reshape(_RT * 32 * 8, 128))

    sid_sc = _prep(student_ids)
    qid_sc = _prep(questions)
    btab = _make_btab(beta_base, beta_gaps)
    t5, a5, b4, l5, p5 = _sc_run(sid_sc, qid_sc, theta_table, alpha_raw, btab)
    theta = jnp.transpose(t5, (2, 4, 0, 1, 3)).reshape(_B, _S, _D)
    alpha = jnp.transpose(a5, (2, 4, 0, 1, 3)).reshape(_B, _S, _D)
    beta = jnp.transpose(b4, (1, 3, 0, 2)).reshape(_B, _S, 4)
    logits = jnp.transpose(l5, (2, 4, 1, 3, 0)).reshape(_B, _S, _K)
    probs = jnp.transpose(p5, (2, 4, 1, 3, 0)).reshape(_B, _S, _K)
    return (theta, alpha, beta, logits, probs)
